# baseline (speedup 1.0000x reference)
import jax
import jax.numpy as jnp
from jax import lax
from jax.experimental import pallas as pl
from jax.experimental.pallas import tpu as pltpu
from jax.experimental.pallas import tpu_sc as plsc

N = 10000
E = 320000
D = 128
NC = 2
NS = 16
NW = NC * NS
K = 128
CPW = 80
NBUF = 2
HALF = CPW // 2
E_PAD = NW * CPW * K
NROWS2D = E_PAD // K
NPAD = N + 16
RSTRIPE = 624


def _striped_rows(sid, copy_fn):
    r0 = sid * RSTRIPE
    for j, nr in enumerate((128, 128, 128, 128, 112)):
        copy_fn(r0 + j * 128, nr)

    @pl.when(sid == NS - 1)
    def _():
        copy_fn(NS * RSTRIPE, N - NS * RSTRIPE)


def _sc_agg_body(h_hbm, src2_hbm, dst2_hbm, zeros_hbm,
                 agg_out,
                 src_blk, dst_blk, rows0, rows1,
                 gsem0, gsem1, agg_sh):
    cid = lax.axis_index("c")
    sid = lax.axis_index("s")
    wid = sid * NC + cid
    rows = (rows0, rows1)
    gsems = (gsem0, gsem1)

    pltpu.sync_copy(zeros_hbm.at[pl.ds(0, K)], rows0)
    _striped_rows(sid, lambda r0, nr: pltpu.sync_copy(
        rows0.at[pl.ds(0, nr)], agg_sh.at[pl.ds(r0, nr)]))
    plsc.subcore_barrier()

    def run_half(half):
        base_row = wid * CPW + half * HALF
        pltpu.sync_copy(src2_hbm.at[pl.ds(base_row, HALF)], src_blk)
        pltpu.sync_copy(dst2_hbm.at[pl.ds(base_row, HALF)], dst_blk)

        def chunk(i, _):
            pltpu.async_copy(h_hbm.at[src_blk.at[i]], rows0, gsem0).wait()
            pltpu.sync_copy(rows0, agg_sh.at[dst_blk.at[i]], add=True)
            return 0

        lax.fori_loop(0, HALF, chunk, 0)

    run_half(0)
    run_half(1)
    plsc.subcore_barrier()

    def out_stripe(r0, nr):
        pltpu.sync_copy(agg_sh.at[pl.ds(r0, nr)], rows0.at[pl.ds(0, nr)])
        pltpu.sync_copy(rows0.at[pl.ds(0, nr)],
                        agg_out.at[pl.ds(cid * N + r0, nr)])

    _striped_rows(sid, out_stripe)


def _sc_deg_body(dst2_hbm, zeros_hbm, ones_hbm,
                 deg_out,
                 dst_blk, ones_v, rows_v, isem, deg_sh):
    cid = lax.axis_index("c")
    sid = lax.axis_index("s")
    wid = sid * NC + cid

    icp_d = pltpu.async_copy(dst2_hbm.at[pl.ds(wid * CPW, CPW)], dst_blk, isem)
    pltpu.sync_copy(zeros_hbm.at[pl.ds(0, K)], rows_v)
    pltpu.sync_copy(ones_hbm.at[pl.ds(0, K)], ones_v)
    _striped_rows(sid, lambda r0, nr: pltpu.sync_copy(
        rows_v.at[pl.ds(0, nr)], deg_sh.at[pl.ds(r0, nr)]))
    icp_d.wait()
    plsc.subcore_barrier()

    def chunk(i, _):
        pltpu.sync_copy(ones_v, deg_sh.at[dst_blk.at[i]], add=True)
        return 0

    lax.fori_loop(0, CPW, chunk, 0)
    plsc.subcore_barrier()

    def out_stripe(r0, nr):
        pltpu.sync_copy(deg_sh.at[pl.ds(r0, nr)], rows_v.at[pl.ds(0, nr)])
        pltpu.sync_copy(rows_v.at[pl.ds(0, nr)],
                        deg_out.at[pl.ds(cid * N + r0, nr)])

    _striped_rows(sid, out_stripe)


_SC_MESH = plsc.VectorSubcoreMesh(core_axis_name="c", subcore_axis_name="s")

_agg_call = pl.kernel(
    _sc_agg_body,
    out_type=jax.ShapeDtypeStruct((NC * N, D), jnp.float32),
    mesh=_SC_MESH,
    scratch_types=[
        pltpu.VMEM((HALF, K), jnp.int32),
        pltpu.VMEM((HALF, K), jnp.int32),
        pltpu.VMEM((K, D), jnp.float32),
        pltpu.VMEM((K, D), jnp.float32),
        pltpu.SemaphoreType.DMA,
        pltpu.SemaphoreType.DMA,
        pltpu.VMEM_SHARED((NPAD, D), jnp.float32),
    ],
)

_deg_call = pl.kernel(
    _sc_deg_body,
    out_type=jax.ShapeDtypeStruct((NC * N, D), jnp.float32),
    mesh=_SC_MESH,
    scratch_types=[
        pltpu.VMEM((CPW, K), jnp.int32),
        pltpu.VMEM((K, D), jnp.float32),
        pltpu.VMEM((K, D), jnp.float32),
        pltpu.SemaphoreType.DMA,
        pltpu.VMEM_SHARED((NPAD, D), jnp.float32),
    ],
)

BN = 1000


def _tc_mlp_body(scale_ref, h_ref, a0_ref, a1_ref, d0_ref, d1_ref,
                 w1_ref, b1_ref, w2_ref, b2_ref, o_ref):
    deg = jnp.maximum(d0_ref[...] + d1_ref[...], 1.0)
    z = scale_ref[0, 0] * h_ref[...] + (a0_ref[...] + a1_ref[...]) / deg
    z = jnp.dot(z, w1_ref[...], preferred_element_type=jnp.float32) + b1_ref[...]
    z = jnp.where(z > 0, z, 0.01 * z)
    z = jnp.dot(z, w2_ref[...], preferred_element_type=jnp.float32) + b2_ref[...]
    o_ref[...] = jnp.where(z > 0, z, 0.01 * z)


_NB = N // BN

_tc_mlp_call = pl.pallas_call(
    _tc_mlp_body,
    grid=(_NB,),
    in_specs=[
        pl.BlockSpec(memory_space=pltpu.SMEM),
        pl.BlockSpec((BN, D), lambda i: (i, 0)),
        pl.BlockSpec((BN, D), lambda i: (i, 0)),
        pl.BlockSpec((BN, D), lambda i: (i + _NB, 0)),
        pl.BlockSpec((BN, D), lambda i: (i, 0)),
        pl.BlockSpec((BN, D), lambda i: (i + _NB, 0)),
        pl.BlockSpec((D, D), lambda i: (0, 0)),
        pl.BlockSpec((1, D), lambda i: (0, 0)),
        pl.BlockSpec((D, D), lambda i: (0, 0)),
        pl.BlockSpec((1, D), lambda i: (0, 0)),
    ],
    out_specs=pl.BlockSpec((BN, D), lambda i: (i, 0)),
    out_shape=jax.ShapeDtypeStruct((N, D), jnp.float32),
)


def kernel(x, edge_index,
           eps0, W1_0, b1_0, W2_0, b2_0,
           eps1, W1_1, b1_1, W2_1, b2_1,
           eps2, W1_2, b1_2, W2_2, b2_2):
    src = edge_index[0]
    dst = edge_index[1]
    npad = E_PAD - E
    src2 = jnp.concatenate(
        [src, jnp.zeros((npad,), jnp.int32)]).reshape(NROWS2D, K)
    dst2 = jnp.concatenate(
        [dst, N + (jnp.arange(npad, dtype=jnp.int32) % 16)]).reshape(NROWS2D, K)
    zeros = jnp.zeros((N, D), jnp.float32)
    ones = jnp.ones((K, D), jnp.float32)

    def mlp(h, agg2, deg2, eps, W1, b1, W2, b2):
        scale = (1.0 + eps).reshape(1, 1)
        return _tc_mlp_call(scale, h, agg2, agg2, deg2, deg2,
                            W1, b1.reshape(1, D), W2, b2.reshape(1, D))

    deg2 = _deg_call(dst2, zeros, ones)
    agg2 = _agg_call(x, src2, dst2, zeros)
    h = mlp(x, agg2, deg2, eps0, W1_0, b1_0, W2_0, b2_0)
    agg2 = _agg_call(h, src2, dst2, zeros)
    h = mlp(h, agg2, deg2, eps1, W1_1, b1_1, W2_1, b2_1)
    agg2 = _agg_call(h, src2, dst2, zeros)
    h = mlp(h, agg2, deg2, eps2, W1_2, b1_2, W2_2, b2_2)
    return h


# speedup vs baseline: 3.4231x; 3.4231x over previous
import jax
import jax.numpy as jnp
from jax import lax
from jax.experimental import pallas as pl
from jax.experimental.pallas import tpu as pltpu
from jax.experimental.pallas import tpu_sc as plsc

N = 10000
E = 320000
D = 128
NC = 2
NS = 16
NW = NC * NS
K = 128
CPW = 80
NBUF = 2
HALF = CPW // 2
E_PAD = NW * CPW * K
NROWS2D = E_PAD // K
NPAD = N + K
RSTRIPE = 624


def _striped_rows(sid, copy_fn):
    r0 = sid * RSTRIPE
    for j, nr in enumerate((128, 128, 128, 128, 112)):
        copy_fn(r0 + j * 128, nr)

    @pl.when(sid == NS - 1)
    def _():
        copy_fn(NS * RSTRIPE, N - NS * RSTRIPE)


def _sc_agg_body(h_hbm, src2_hbm, dst2_hbm, zeros_hbm,
                 agg_out,
                 src_blk, dst_blk, rows0, rows1,
                 gsem0, gsem1, agg_sh):
    cid = lax.axis_index("c")
    sid = lax.axis_index("s")
    wid = sid * NC + cid
    rows = (rows0, rows1)
    gsems = (gsem0, gsem1)

    pltpu.sync_copy(zeros_hbm.at[pl.ds(0, K)], rows0)
    _striped_rows(sid, lambda r0, nr: pltpu.sync_copy(
        rows0.at[pl.ds(0, nr)], agg_sh.at[pl.ds(r0, nr)]))
    plsc.subcore_barrier()

    def run_half(half):
        base_row = wid * CPW + half * HALF
        pltpu.sync_copy(src2_hbm.at[pl.ds(base_row, HALF)], src_blk)
        pltpu.sync_copy(dst2_hbm.at[pl.ds(base_row, HALF)], dst_blk)
        for b in range(NBUF):
            pltpu.async_copy(h_hbm.at[src_blk.at[b]], rows[b], gsems[b])

        def outer(g, _):
            for b in range(NBUF):
                i = NBUF * g + b
                pltpu.make_async_copy(h_hbm.at[src_blk.at[b]],
                                      rows[b], gsems[b]).wait()
                pltpu.sync_copy(rows[b], agg_sh.at[dst_blk.at[i]], add=True)

                @pl.when(g < HALF // NBUF - 1)
                def _():
                    pltpu.async_copy(h_hbm.at[src_blk.at[i + NBUF]],
                                     rows[b], gsems[b])
            return 0

        lax.fori_loop(0, HALF // NBUF, outer, 0)

    run_half(0)
    run_half(1)
    plsc.subcore_barrier()

    def out_stripe(r0, nr):
        pltpu.sync_copy(agg_sh.at[pl.ds(r0, nr)], rows0.at[pl.ds(0, nr)])
        pltpu.sync_copy(rows0.at[pl.ds(0, nr)],
                        agg_out.at[pl.ds(cid * N + r0, nr)])

    _striped_rows(sid, out_stripe)


def _sc_deg_body(dst2_hbm, zeros_hbm, ones_hbm,
                 deg_out,
                 dst_blk, ones_v, rows_v, isem, deg_sh):
    cid = lax.axis_index("c")
    sid = lax.axis_index("s")
    wid = sid * NC + cid

    icp_d = pltpu.async_copy(dst2_hbm.at[pl.ds(wid * CPW, CPW)], dst_blk, isem)
    pltpu.sync_copy(zeros_hbm.at[pl.ds(0, K)], rows_v)
    pltpu.sync_copy(ones_hbm.at[pl.ds(0, K)], ones_v)
    _striped_rows(sid, lambda r0, nr: pltpu.sync_copy(
        rows_v.at[pl.ds(0, nr)], deg_sh.at[pl.ds(r0, nr)]))
    icp_d.wait()
    plsc.subcore_barrier()

    def chunk(i, _):
        pltpu.sync_copy(ones_v, deg_sh.at[dst_blk.at[i]], add=True)
        return 0

    lax.fori_loop(0, CPW, chunk, 0)
    plsc.subcore_barrier()

    def out_stripe(r0, nr):
        pltpu.sync_copy(deg_sh.at[pl.ds(r0, nr)], rows_v.at[pl.ds(0, nr)])
        pltpu.sync_copy(rows_v.at[pl.ds(0, nr)],
                        deg_out.at[pl.ds(cid * N + r0, nr)])

    _striped_rows(sid, out_stripe)


_SC_MESH = plsc.VectorSubcoreMesh(core_axis_name="c", subcore_axis_name="s")

_agg_call = pl.kernel(
    _sc_agg_body,
    out_type=jax.ShapeDtypeStruct((NC * N, D), jnp.float32),
    mesh=_SC_MESH,
    scratch_types=[
        pltpu.VMEM((HALF, K), jnp.int32),
        pltpu.VMEM((HALF, K), jnp.int32),
        pltpu.VMEM((K, D), jnp.float32),
        pltpu.VMEM((K, D), jnp.float32),
        pltpu.SemaphoreType.DMA,
        pltpu.SemaphoreType.DMA,
        pltpu.VMEM_SHARED((NPAD, D), jnp.float32),
    ],
)

_deg_call = pl.kernel(
    _sc_deg_body,
    out_type=jax.ShapeDtypeStruct((NC * N, D), jnp.float32),
    mesh=_SC_MESH,
    scratch_types=[
        pltpu.VMEM((CPW, K), jnp.int32),
        pltpu.VMEM((K, D), jnp.float32),
        pltpu.VMEM((K, D), jnp.float32),
        pltpu.SemaphoreType.DMA,
        pltpu.VMEM_SHARED((NPAD, D), jnp.float32),
    ],
)

BN = 1000


def _tc_mlp_body(scale_ref, h_ref, a0_ref, a1_ref, d0_ref, d1_ref,
                 w1_ref, b1_ref, w2_ref, b2_ref, o_ref):
    deg = jnp.maximum(d0_ref[...] + d1_ref[...], 1.0)
    z = scale_ref[0, 0] * h_ref[...] + (a0_ref[...] + a1_ref[...]) / deg
    z = jnp.dot(z, w1_ref[...], preferred_element_type=jnp.float32) + b1_ref[...]
    z = jnp.where(z > 0, z, 0.01 * z)
    z = jnp.dot(z, w2_ref[...], preferred_element_type=jnp.float32) + b2_ref[...]
    o_ref[...] = jnp.where(z > 0, z, 0.01 * z)


_NB = N // BN

_tc_mlp_call = pl.pallas_call(
    _tc_mlp_body,
    grid=(_NB,),
    in_specs=[
        pl.BlockSpec(memory_space=pltpu.SMEM),
        pl.BlockSpec((BN, D), lambda i: (i, 0)),
        pl.BlockSpec((BN, D), lambda i: (i, 0)),
        pl.BlockSpec((BN, D), lambda i: (i + _NB, 0)),
        pl.BlockSpec((BN, D), lambda i: (i, 0)),
        pl.BlockSpec((BN, D), lambda i: (i + _NB, 0)),
        pl.BlockSpec((D, D), lambda i: (0, 0)),
        pl.BlockSpec((1, D), lambda i: (0, 0)),
        pl.BlockSpec((D, D), lambda i: (0, 0)),
        pl.BlockSpec((1, D), lambda i: (0, 0)),
    ],
    out_specs=pl.BlockSpec((BN, D), lambda i: (i, 0)),
    out_shape=jax.ShapeDtypeStruct((N, D), jnp.float32),
)


def kernel(x, edge_index,
           eps0, W1_0, b1_0, W2_0, b2_0,
           eps1, W1_1, b1_1, W2_1, b2_1,
           eps2, W1_2, b1_2, W2_2, b2_2):
    src = edge_index[0]
    dst = edge_index[1]
    npad = E_PAD - E
    pad_iota = jnp.arange(npad, dtype=jnp.int32)
    src2 = jnp.concatenate([src, pad_iota % N]).reshape(NROWS2D, K)
    dst2 = jnp.concatenate([dst, N + pad_iota % K]).reshape(NROWS2D, K)
    zeros = jnp.zeros((N, D), jnp.float32)
    ones = jnp.ones((K, D), jnp.float32)

    def mlp(h, agg2, deg2, eps, W1, b1, W2, b2):
        scale = (1.0 + eps).reshape(1, 1)
        return _tc_mlp_call(scale, h, agg2, agg2, deg2, deg2,
                            W1, b1.reshape(1, D), W2, b2.reshape(1, D))

    deg2 = _deg_call(dst2, zeros, ones)
    agg2 = _agg_call(x, src2, dst2, zeros)
    h = mlp(x, agg2, deg2, eps0, W1_0, b1_0, W2_0, b2_0)
    agg2 = _agg_call(h, src2, dst2, zeros)
    h = mlp(h, agg2, deg2, eps1, W1_1, b1_1, W2_1, b2_1)
    agg2 = _agg_call(h, src2, dst2, zeros)
    h = mlp(h, agg2, deg2, eps2, W1_2, b1_2, W2_2, b2_2)
    return h


# speedup vs baseline: 3.4724x; 1.0144x over previous
import jax
import jax.numpy as jnp
from jax import lax
from jax.experimental import pallas as pl
from jax.experimental.pallas import tpu as pltpu
from jax.experimental.pallas import tpu_sc as plsc

N = 10000
E = 320000
D = 128
NC = 2
NS = 16
NW = NC * NS
K = 128
CPW = 80
NBUF = 2
NSTAGE = 5
STAGE = CPW // NSTAGE
E_PAD = NW * CPW * K
NROWS2D = E_PAD // K
NPAD = N + K
RSTRIPE = 624


def _striped_rows(sid, copy_fn):
    r0 = sid * RSTRIPE
    for j, nr in enumerate((128, 128, 128, 128, 112)):
        copy_fn(r0 + j * 128, nr)

    @pl.when(sid == NS - 1)
    def _():
        copy_fn(NS * RSTRIPE, N - NS * RSTRIPE)


def _sc_agg_body(h_hbm, src2_hbm, dst2_hbm, zeros_hbm,
                 agg_out,
                 src_a, dst_a, src_b, dst_b, rows0, rows1,
                 gsem0, gsem1, isem, agg_sh):
    cid = lax.axis_index("c")
    sid = lax.axis_index("s")
    wid = sid * NC + cid
    rows = (rows0, rows1)
    gsems = (gsem0, gsem1)
    blksets = ((src_a, dst_a), (src_b, dst_b))

    i0s = pltpu.async_copy(src2_hbm.at[pl.ds(wid * CPW, STAGE)], src_a, isem)
    i0d = pltpu.async_copy(dst2_hbm.at[pl.ds(wid * CPW, STAGE)], dst_a, isem)
    pltpu.sync_copy(zeros_hbm.at[pl.ds(0, K)], rows0)
    _striped_rows(sid, lambda r0, nr: pltpu.sync_copy(
        rows0.at[pl.ds(0, nr)], agg_sh.at[pl.ds(r0, nr)]))
    i0s.wait()
    i0d.wait()
    plsc.subcore_barrier()

    for b in range(NBUF):
        pltpu.async_copy(h_hbm.at[src_a.at[b]], rows[b], gsems[b])

    for q in range(NSTAGE):
        src_c, dst_c = blksets[q % 2]
        if q + 1 < NSTAGE:
            src_n, dst_n = blksets[(q + 1) % 2]
            base_n = wid * CPW + (q + 1) * STAGE
            ins = pltpu.async_copy(src2_hbm.at[pl.ds(base_n, STAGE)],
                                   src_n, isem)
            ind = pltpu.async_copy(dst2_hbm.at[pl.ds(base_n, STAGE)],
                                   dst_n, isem)

        def mid(g, _, src_c=src_c, dst_c=dst_c):
            for b in range(NBUF):
                i = NBUF * g + b
                pltpu.make_async_copy(h_hbm.at[src_c.at[b]],
                                      rows[b], gsems[b]).wait()
                pltpu.sync_copy(rows[b], agg_sh.at[dst_c.at[i]], add=True)
                pltpu.async_copy(h_hbm.at[src_c.at[i + NBUF]],
                                 rows[b], gsems[b])
            return 0

        lax.fori_loop(0, STAGE // NBUF - 1, mid, 0)

        if q + 1 < NSTAGE:
            ins.wait()
            ind.wait()
        for b in range(NBUF):
            i = STAGE - NBUF + b
            pltpu.make_async_copy(h_hbm.at[src_c.at[b]],
                                  rows[b], gsems[b]).wait()
            pltpu.sync_copy(rows[b], agg_sh.at[dst_c.at[i]], add=True)
            if q + 1 < NSTAGE:
                pltpu.async_copy(h_hbm.at[src_n.at[b]], rows[b], gsems[b])
    plsc.subcore_barrier()

    def out_stripe(r0, nr):
        pltpu.sync_copy(agg_sh.at[pl.ds(r0, nr)], rows0.at[pl.ds(0, nr)])
        pltpu.sync_copy(rows0.at[pl.ds(0, nr)],
                        agg_out.at[pl.ds(cid * N + r0, nr)])

    _striped_rows(sid, out_stripe)


def _sc_deg_body(dst2_hbm, zeros_hbm, ones_hbm,
                 deg_out,
                 dst_blk, ones_v, rows_v, isem, deg_sh):
    cid = lax.axis_index("c")
    sid = lax.axis_index("s")
    wid = sid * NC + cid

    icp_d = pltpu.async_copy(dst2_hbm.at[pl.ds(wid * CPW, CPW)], dst_blk, isem)
    pltpu.sync_copy(zeros_hbm.at[pl.ds(0, K)], rows_v)
    pltpu.sync_copy(ones_hbm.at[pl.ds(0, K)], ones_v)
    _striped_rows(sid, lambda r0, nr: pltpu.sync_copy(
        rows_v.at[pl.ds(0, nr)], deg_sh.at[pl.ds(r0, nr)]))
    icp_d.wait()
    plsc.subcore_barrier()

    def chunk(i, _):
        pltpu.sync_copy(ones_v, deg_sh.at[dst_blk.at[i]], add=True)
        return 0

    lax.fori_loop(0, CPW, chunk, 0)
    plsc.subcore_barrier()

    def out_stripe(r0, nr):
        pltpu.sync_copy(deg_sh.at[pl.ds(r0, nr)], rows_v.at[pl.ds(0, nr)])
        pltpu.sync_copy(rows_v.at[pl.ds(0, nr)],
                        deg_out.at[pl.ds(cid * N + r0, nr)])

    _striped_rows(sid, out_stripe)


_SC_MESH = plsc.VectorSubcoreMesh(core_axis_name="c", subcore_axis_name="s")

_agg_call = pl.kernel(
    _sc_agg_body,
    out_type=jax.ShapeDtypeStruct((NC * N, D), jnp.float32),
    mesh=_SC_MESH,
    scratch_types=[
        pltpu.VMEM((STAGE, K), jnp.int32),
        pltpu.VMEM((STAGE, K), jnp.int32),
        pltpu.VMEM((STAGE, K), jnp.int32),
        pltpu.VMEM((STAGE, K), jnp.int32),
        pltpu.VMEM((K, D), jnp.float32),
        pltpu.VMEM((K, D), jnp.float32),
        pltpu.SemaphoreType.DMA,
        pltpu.SemaphoreType.DMA,
        pltpu.SemaphoreType.DMA,
        pltpu.VMEM_SHARED((NPAD, D), jnp.float32),
    ],
)

_deg_call = pl.kernel(
    _sc_deg_body,
    out_type=jax.ShapeDtypeStruct((NC * N, D), jnp.float32),
    mesh=_SC_MESH,
    scratch_types=[
        pltpu.VMEM((CPW, K), jnp.int32),
        pltpu.VMEM((K, D), jnp.float32),
        pltpu.VMEM((K, D), jnp.float32),
        pltpu.SemaphoreType.DMA,
        pltpu.VMEM_SHARED((NPAD, D), jnp.float32),
    ],
)

BN = 1000


def _tc_mlp_body(scale_ref, h_ref, a0_ref, a1_ref, d0_ref, d1_ref,
                 w1_ref, b1_ref, w2_ref, b2_ref, o_ref):
    deg = jnp.maximum(d0_ref[...] + d1_ref[...], 1.0)
    z = scale_ref[0, 0] * h_ref[...] + (a0_ref[...] + a1_ref[...]) / deg
    z = jnp.dot(z, w1_ref[...], preferred_element_type=jnp.float32) + b1_ref[...]
    z = jnp.where(z > 0, z, 0.01 * z)
    z = jnp.dot(z, w2_ref[...], preferred_element_type=jnp.float32) + b2_ref[...]
    o_ref[...] = jnp.where(z > 0, z, 0.01 * z)


_NB = N // BN

_tc_mlp_call = pl.pallas_call(
    _tc_mlp_body,
    grid=(_NB,),
    in_specs=[
        pl.BlockSpec(memory_space=pltpu.SMEM),
        pl.BlockSpec((BN, D), lambda i: (i, 0)),
        pl.BlockSpec((BN, D), lambda i: (i, 0)),
        pl.BlockSpec((BN, D), lambda i: (i + _NB, 0)),
        pl.BlockSpec((BN, D), lambda i: (i, 0)),
        pl.BlockSpec((BN, D), lambda i: (i + _NB, 0)),
        pl.BlockSpec((D, D), lambda i: (0, 0)),
        pl.BlockSpec((1, D), lambda i: (0, 0)),
        pl.BlockSpec((D, D), lambda i: (0, 0)),
        pl.BlockSpec((1, D), lambda i: (0, 0)),
    ],
    out_specs=pl.BlockSpec((BN, D), lambda i: (i, 0)),
    out_shape=jax.ShapeDtypeStruct((N, D), jnp.float32),
)


def kernel(x, edge_index,
           eps0, W1_0, b1_0, W2_0, b2_0,
           eps1, W1_1, b1_1, W2_1, b2_1,
           eps2, W1_2, b1_2, W2_2, b2_2):
    src = edge_index[0]
    dst = edge_index[1]
    npad = E_PAD - E
    pad_iota = jnp.arange(npad, dtype=jnp.int32)
    src2 = jnp.concatenate([src, pad_iota % N]).reshape(NROWS2D, K)
    dst2 = jnp.concatenate([dst, N + pad_iota % K]).reshape(NROWS2D, K)
    zeros = jnp.zeros((N, D), jnp.float32)

    def mlp(h, agg2, deg2, eps, W1, b1, W2, b2):
        scale = (1.0 + eps).reshape(1, 1)
        return _tc_mlp_call(scale, h, agg2, agg2, deg2, deg2,
                            W1, b1.reshape(1, D), W2, b2.reshape(1, D))

    ones = jnp.ones((K, D), jnp.float32)
    deg2 = _deg_call(dst2, zeros, ones)
    agg2 = _agg_call(x, src2, dst2, zeros)
    h = mlp(x, agg2, deg2, eps0, W1_0, b1_0, W2_0, b2_0)
    agg2 = _agg_call(h, src2, dst2, zeros)
    h = mlp(h, agg2, deg2, eps1, W1_1, b1_1, W2_1, b2_1)
    agg2 = _agg_call(h, src2, dst2, zeros)
    h = mlp(h, agg2, deg2, eps2, W1_2, b1_2, W2_2, b2_2)
    return h


# speedup vs baseline: 3.7670x; 1.0849x over previous
import jax
import jax.numpy as jnp
from jax import lax
from jax.experimental import pallas as pl
from jax.experimental.pallas import tpu as pltpu
from jax.experimental.pallas import tpu_sc as plsc

N = 10000
E = 320000
D = 128
NC = 2
NS = 16
NW = NC * NS
K = 128
CPW = 80
CH = 64
NBUF = 4
NSTAGE = 5
E_PAD = NW * CPW * K
CPWC = E_PAD // (NW * CH)
STAGE = CPWC // NSTAGE
NROWS2D = E_PAD // K
NROWSC = E_PAD // CH
NPAD = N + K
RSTRIPE = 624


def _striped_rows(sid, copy_fn):
    r0 = sid * RSTRIPE
    off = 0
    for nr in (64,) * 9 + (48,):
        copy_fn(r0 + off, nr)
        off += nr

    @pl.when(sid == NS - 1)
    def _():
        copy_fn(NS * RSTRIPE, N - NS * RSTRIPE)


def _sc_agg_body(h_hbm, src2_hbm, dst2_hbm, zeros_hbm,
                 agg_out,
                 src_a, dst_a, src_b, dst_b, rows0, rows1, rows2, rows3,
                 gsem0, gsem1, gsem2, gsem3, isem, agg_sh):
    cid = lax.axis_index("c")
    sid = lax.axis_index("s")
    wid = sid * NC + cid
    rows = (rows0, rows1, rows2, rows3)
    gsems = (gsem0, gsem1, gsem2, gsem3)
    blksets = ((src_a, dst_a), (src_b, dst_b))

    i0s = pltpu.async_copy(src2_hbm.at[pl.ds(wid * CPWC, STAGE)], src_a, isem)
    i0d = pltpu.async_copy(dst2_hbm.at[pl.ds(wid * CPWC, STAGE)], dst_a, isem)
    pltpu.sync_copy(zeros_hbm.at[pl.ds(0, CH)], rows0)
    _striped_rows(sid, lambda r0, nr: pltpu.sync_copy(
        rows0.at[pl.ds(0, nr)], agg_sh.at[pl.ds(r0, nr)]))
    i0s.wait()
    i0d.wait()
    plsc.subcore_barrier()

    for b in range(NBUF):
        pltpu.async_copy(h_hbm.at[src_a.at[b]], rows[b], gsems[b])

    for q in range(NSTAGE):
        src_c, dst_c = blksets[q % 2]
        if q + 1 < NSTAGE:
            src_n, dst_n = blksets[(q + 1) % 2]
            base_n = wid * CPWC + (q + 1) * STAGE
            ins = pltpu.async_copy(src2_hbm.at[pl.ds(base_n, STAGE)],
                                   src_n, isem)
            ind = pltpu.async_copy(dst2_hbm.at[pl.ds(base_n, STAGE)],
                                   dst_n, isem)

        def mid(g, _, src_c=src_c, dst_c=dst_c):
            for b in range(NBUF):
                i = NBUF * g + b
                pltpu.make_async_copy(h_hbm.at[src_c.at[b]],
                                      rows[b], gsems[b]).wait()
                pltpu.sync_copy(rows[b], agg_sh.at[dst_c.at[i]], add=True)
                pltpu.async_copy(h_hbm.at[src_c.at[i + NBUF]],
                                 rows[b], gsems[b])
            return 0

        lax.fori_loop(0, STAGE // NBUF - 1, mid, 0)

        if q + 1 < NSTAGE:
            ins.wait()
            ind.wait()
        for b in range(NBUF):
            i = STAGE - NBUF + b
            pltpu.make_async_copy(h_hbm.at[src_c.at[b]],
                                  rows[b], gsems[b]).wait()
            pltpu.sync_copy(rows[b], agg_sh.at[dst_c.at[i]], add=True)
            if q + 1 < NSTAGE:
                pltpu.async_copy(h_hbm.at[src_n.at[b]], rows[b], gsems[b])
    plsc.subcore_barrier()

    def out_stripe(r0, nr):
        pltpu.sync_copy(agg_sh.at[pl.ds(r0, nr)], rows0.at[pl.ds(0, nr)])
        pltpu.sync_copy(rows0.at[pl.ds(0, nr)],
                        agg_out.at[pl.ds(cid * N + r0, nr)])

    _striped_rows(sid, out_stripe)


def _sc_deg_body(dst2_hbm, zeros_hbm, ones_hbm,
                 deg_out,
                 dst_blk, ones_v, rows_v, isem, deg_sh):
    cid = lax.axis_index("c")
    sid = lax.axis_index("s")
    wid = sid * NC + cid

    icp_d = pltpu.async_copy(dst2_hbm.at[pl.ds(wid * CPW, CPW)], dst_blk, isem)
    pltpu.sync_copy(zeros_hbm.at[pl.ds(0, K)], rows_v)
    pltpu.sync_copy(ones_hbm.at[pl.ds(0, K)], ones_v)
    _striped_rows(sid, lambda r0, nr: pltpu.sync_copy(
        rows_v.at[pl.ds(0, nr)], deg_sh.at[pl.ds(r0, nr)]))
    icp_d.wait()
    plsc.subcore_barrier()

    def chunk(i, _):
        pltpu.sync_copy(ones_v, deg_sh.at[dst_blk.at[i]], add=True)
        return 0

    lax.fori_loop(0, CPW, chunk, 0)
    plsc.subcore_barrier()

    def out_stripe(r0, nr):
        pltpu.sync_copy(deg_sh.at[pl.ds(r0, nr)], rows_v.at[pl.ds(0, nr)])
        pltpu.sync_copy(rows_v.at[pl.ds(0, nr)],
                        deg_out.at[pl.ds(cid * N + r0, nr)])

    _striped_rows(sid, out_stripe)


_SC_MESH = plsc.VectorSubcoreMesh(core_axis_name="c", subcore_axis_name="s")

_agg_call = pl.kernel(
    _sc_agg_body,
    out_type=jax.ShapeDtypeStruct((NC * N, D), jnp.float32),
    mesh=_SC_MESH,
    scratch_types=[
        pltpu.VMEM((STAGE, CH), jnp.int32),
        pltpu.VMEM((STAGE, CH), jnp.int32),
        pltpu.VMEM((STAGE, CH), jnp.int32),
        pltpu.VMEM((STAGE, CH), jnp.int32),
        pltpu.VMEM((CH, D), jnp.float32),
        pltpu.VMEM((CH, D), jnp.float32),
        pltpu.VMEM((CH, D), jnp.float32),
        pltpu.VMEM((CH, D), jnp.float32),
        pltpu.SemaphoreType.DMA,
        pltpu.SemaphoreType.DMA,
        pltpu.SemaphoreType.DMA,
        pltpu.SemaphoreType.DMA,
        pltpu.SemaphoreType.DMA,
        pltpu.VMEM_SHARED((NPAD, D), jnp.float32),
    ],
)

_deg_call = pl.kernel(
    _sc_deg_body,
    out_type=jax.ShapeDtypeStruct((NC * N, D), jnp.float32),
    mesh=_SC_MESH,
    scratch_types=[
        pltpu.VMEM((CPW, K), jnp.int32),
        pltpu.VMEM((K, D), jnp.float32),
        pltpu.VMEM((K, D), jnp.float32),
        pltpu.SemaphoreType.DMA,
        pltpu.VMEM_SHARED((NPAD, D), jnp.float32),
    ],
)

BN = 1000


def _tc_mlp_body(scale_ref, h_ref, a0_ref, a1_ref, d0_ref, d1_ref,
                 w1_ref, b1_ref, w2_ref, b2_ref, o_ref):
    deg = jnp.maximum(d0_ref[...] + d1_ref[...], 1.0)
    z = scale_ref[0, 0] * h_ref[...] + (a0_ref[...] + a1_ref[...]) / deg
    z = jnp.dot(z, w1_ref[...], preferred_element_type=jnp.float32) + b1_ref[...]
    z = jnp.where(z > 0, z, 0.01 * z)
    z = jnp.dot(z, w2_ref[...], preferred_element_type=jnp.float32) + b2_ref[...]
    o_ref[...] = jnp.where(z > 0, z, 0.01 * z)


_NB = N // BN

_tc_mlp_call = pl.pallas_call(
    _tc_mlp_body,
    grid=(_NB,),
    in_specs=[
        pl.BlockSpec(memory_space=pltpu.SMEM),
        pl.BlockSpec((BN, D), lambda i: (i, 0)),
        pl.BlockSpec((BN, D), lambda i: (i, 0)),
        pl.BlockSpec((BN, D), lambda i: (i + _NB, 0)),
        pl.BlockSpec((BN, D), lambda i: (i, 0)),
        pl.BlockSpec((BN, D), lambda i: (i + _NB, 0)),
        pl.BlockSpec((D, D), lambda i: (0, 0)),
        pl.BlockSpec((1, D), lambda i: (0, 0)),
        pl.BlockSpec((D, D), lambda i: (0, 0)),
        pl.BlockSpec((1, D), lambda i: (0, 0)),
    ],
    out_specs=pl.BlockSpec((BN, D), lambda i: (i, 0)),
    out_shape=jax.ShapeDtypeStruct((N, D), jnp.float32),
)


def kernel(x, edge_index,
           eps0, W1_0, b1_0, W2_0, b2_0,
           eps1, W1_1, b1_1, W2_1, b2_1,
           eps2, W1_2, b1_2, W2_2, b2_2):
    src = edge_index[0]
    dst = edge_index[1]
    npad = E_PAD - E
    pad_iota = jnp.arange(npad, dtype=jnp.int32)
    src_p = jnp.concatenate([src, pad_iota % N])
    dst_p = jnp.concatenate([dst, N + pad_iota % K])
    src2c = src_p.reshape(NROWSC, CH)
    dst2c = dst_p.reshape(NROWSC, CH)
    dst2 = dst_p.reshape(NROWS2D, K)
    zeros = jnp.zeros((N, D), jnp.float32)

    def mlp(h, agg2, deg2, eps, W1, b1, W2, b2):
        scale = (1.0 + eps).reshape(1, 1)
        return _tc_mlp_call(scale, h, agg2, agg2, deg2, deg2,
                            W1, b1.reshape(1, D), W2, b2.reshape(1, D))

    ones = jnp.ones((K, D), jnp.float32)
    deg2 = _deg_call(dst2, zeros, ones)
    agg2 = _agg_call(x, src2c, dst2c, zeros)
    h = mlp(x, agg2, deg2, eps0, W1_0, b1_0, W2_0, b2_0)
    agg2 = _agg_call(h, src2c, dst2c, zeros)
    h = mlp(h, agg2, deg2, eps1, W1_1, b1_1, W2_1, b2_1)
    agg2 = _agg_call(h, src2c, dst2c, zeros)
    h = mlp(h, agg2, deg2, eps2, W1_2, b1_2, W2_2, b2_2)
    return h
